# baseline (speedup 1.0000x reference)
import functools

import jax
import jax.numpy as jnp
from jax import lax
from jax.experimental import pallas as pl
from jax.experimental.pallas import tpu as pltpu
from jax.experimental.pallas import tpu_sc as plsc

_B, _L = 64, 2048
_V = 256
_MIN_KEPT = 4

_info = plsc.get_sparse_core_info()
_NC, _NS, _LANES = _info.num_cores, _info.num_subcores, _info.num_lanes
_NW = _NC * _NS
_ROWS_PER_W = _B // _NW
_CHUNKS = _L // _LANES
_HB = _LANES * _V
_WL = _ROWS_PER_W * _L
_WH = _ROWS_PER_W * _HB


def _tables_body(emb_ref, w1_ref, b1_ref, w2_ref, b2_ref, logit_ref):
    h = jnp.dot(emb_ref[...], w1_ref[...], preferred_element_type=jnp.float32)
    h = jnp.maximum(h + b1_ref[...], 0.0)
    logit = jnp.dot(h, w2_ref[...], preferred_element_type=jnp.float32)
    logit_ref[...] = logit + b2_ref[0, 0]


def _masker_tables(emb_m, W1, b1, w2, b2):
    return pl.pallas_call(
        _tables_body,
        out_shape=jax.ShapeDtypeStruct((_V, 1), jnp.float32),
    )(emb_m, W1, b1.reshape(1, -1), w2.reshape(-1, 1), b2.reshape(1, 1))


def _sc_body(ids_hbm, lt_hbm, pt_hbm, zeros_hbm,
             ml_hbm, pp_hbm, h16_hbm,
             ids_v, tab_v, ml_v, pp_v, h16_v,
             sem_misc, sem_ids, sem_out):
    wid = lax.axis_index("s") * _NC + lax.axis_index("c")

    cp_tab = pltpu.async_copy(lt_hbm, tab_v.at[pl.ds(0, _V)], sem_misc)
    cp_tab2 = pltpu.async_copy(pt_hbm, tab_v.at[pl.ds(_V, _V)], sem_misc)
    cp_zero = pltpu.async_copy(zeros_hbm, h16_v, sem_misc)
    cp_ids = pltpu.async_copy(ids_hbm.at[wid], ids_v, sem_ids)
    cp_tab.wait()
    cp_tab2.wait()
    cp_zero.wait()
    cp_ids.wait()

    lane_off = lax.iota(jnp.int32, _LANES) * _V
    ones = jnp.full((_LANES,), 1.0, jnp.float32)
    for r in range(_ROWS_PER_W):
        ib, hb = r * _L, r * _HB
        off = lane_off + hb

        def gather_body(i, _):
            for u in range(8):
                base = ib + (i * 8 + u) * _LANES
                idx = ids_v[pl.ds(base, _LANES)]
                ml_v[pl.ds(base, _LANES)] = plsc.load_gather(tab_v, [idx])
                pp_v[pl.ds(base, _LANES)] = plsc.load_gather(
                    tab_v, [idx + _V])
            return 0
        lax.fori_loop(0, _CHUNKS // 8, gather_body, 0)

        def hist_body(j, _):
            for u in range(8):
                base = ib + (j * 8 + u) * _LANES
                idx = ids_v[pl.ds(base, _LANES)]
                plsc.addupdate_scatter(h16_v, [idx + off], ones)
            return 0
        lax.fori_loop(0, _CHUNKS // 8, hist_body, 0)

    cps = [pltpu.async_copy(ml_v, ml_hbm.at[wid], sem_out),
           pltpu.async_copy(pp_v, pp_hbm.at[wid], sem_out),
           pltpu.async_copy(h16_v, h16_hbm.at[wid], sem_out)]
    for cp in cps:
        cp.wait()


_sc_stage = functools.partial(
    pl.kernel,
    out_type=[
        jax.ShapeDtypeStruct((_NW, _WL), jnp.float32),
        jax.ShapeDtypeStruct((_NW, _WL), jnp.float32),
        jax.ShapeDtypeStruct((_NW, _WH), jnp.float32),
    ],
    mesh=plsc.VectorSubcoreMesh(core_axis_name="c", subcore_axis_name="s"),
    compiler_params=pltpu.CompilerParams(needs_layout_passes=False),
    scratch_types=[
        pltpu.VMEM((_WL,), jnp.int32),
        pltpu.VMEM((2 * _V,), jnp.float32),
        pltpu.VMEM((_WL,), jnp.float32),
        pltpu.VMEM((_WL,), jnp.float32),
        pltpu.VMEM((_WH,), jnp.float32),
        pltpu.SemaphoreType.DMA,
        pltpu.SemaphoreType.DMA,
        pltpu.SemaphoreType.DMA,
    ],
)(_sc_body)


def _final_body(h16_ref, keep_ref, pc_ref, pr_ref, embv_ref, wcls_ref, bcls_ref,
                cls_ref, len_ref):
    hist = h16_ref[:, 0:_V]
    for l in range(1, _LANES):
        hist = hist + h16_ref[:, l * _V:(l + 1) * _V]
    kept = hist * keep_ref[...]
    count = jnp.sum(kept, axis=1, keepdims=True)
    pc = pc_ref[...]
    pr = pr_ref[...]
    iu = lax.broadcasted_iota(jnp.int32, (_V, _V), 0)
    iv = lax.broadcasted_iota(jnp.int32, (_V, _V), 1)
    before = jnp.where((pc > pr) | ((pc == pr) & (iu < iv)), 1.0, 0.0)
    cum = jnp.dot(hist, before, preferred_element_type=jnp.float32,
                  precision=lax.Precision.HIGHEST)
    topk_take = jnp.minimum(jnp.maximum(float(_MIN_KEPT) - cum, 0.0), hist)
    use_fb = count < float(_MIN_KEPT)
    w = jnp.where(use_fb, topk_take, kept)
    pooled_sum = jnp.dot(w, embv_ref[...], preferred_element_type=jnp.float32,
                         precision=lax.Precision.HIGHEST)
    lengths = jnp.where(use_fb, float(_MIN_KEPT), count)
    pooled = pooled_sum / jnp.maximum(lengths, 1.0)
    cls = jnp.dot(pooled, wcls_ref[...], preferred_element_type=jnp.float32)
    cls_ref[...] = cls + bcls_ref[0, 0]
    len_ref[...] = lengths.astype(jnp.int32)


def _finalize(h16, keep_row, prob_col, prob_row, emb_v_tail, W_cls_tail, b_cls):
    return pl.pallas_call(
        _final_body,
        out_shape=[
            jax.ShapeDtypeStruct((_B, 1), jnp.float32),
            jax.ShapeDtypeStruct((_B, 1), jnp.int32),
        ],
    )(h16, keep_row, prob_col, prob_row, emb_v_tail, W_cls_tail, b_cls)


def kernel(byte_ids, emb_m, W1, b1, w2, b2, emb_v, W_cls, b_cls):
    ids = jnp.asarray(byte_ids).astype(jnp.int32).reshape(_NW, _WL)
    logit_col = _masker_tables(emb_m, W1, b1, w2, b2)
    logit_tab = logit_col.reshape(_V)
    prob_tab = jax.nn.sigmoid(logit_tab)
    keep_row = (prob_tab > 0.5).astype(jnp.float32).reshape(1, _V)
    zeros_hb = jnp.zeros((_WH,), jnp.float32)

    ml, pp, h16 = _sc_stage(ids, logit_tab, prob_tab, zeros_hb)
    mask_logits = ml.reshape(_B, _L)
    prune_probs = pp.reshape(_B, _L)

    cls, lengths = _finalize(
        h16.reshape(_B, _HB), keep_row, prob_tab.reshape(_V, 1),
        prob_tab.reshape(1, _V), emb_v, W_cls[2:, :], b_cls.reshape(1, 1))
    return mask_logits, prune_probs, cls, lengths.reshape(_B)


# speedup vs baseline: 1.1959x; 1.1959x over previous
import functools

import jax
import jax.numpy as jnp
from jax import lax
from jax.experimental import pallas as pl
from jax.experimental.pallas import tpu as pltpu
from jax.experimental.pallas import tpu_sc as plsc

_B, _L = 64, 2048
_V = 256
_MIN_KEPT = 4

_info = plsc.get_sparse_core_info()
_NC, _NS, _LANES = _info.num_cores, _info.num_subcores, _info.num_lanes
_NW = _NC * _NS
_ROWS_PER_W = _B // _NW
_CHUNKS = _L // _LANES
_HB = _LANES * _V
_WL = _ROWS_PER_W * _L
_WH = _ROWS_PER_W * _HB


def _tables_body(emb_ref, w1_ref, b1_ref, w2_ref, b2_ref, logit_ref):
    h = jnp.dot(emb_ref[...], w1_ref[...], preferred_element_type=jnp.float32)
    h = jnp.maximum(h + b1_ref[...], 0.0)
    logit = jnp.dot(h, w2_ref[...], preferred_element_type=jnp.float32)
    logit_ref[...] = logit + b2_ref[0, 0]


def _masker_tables(emb_m, W1, b1, w2, b2):
    return pl.pallas_call(
        _tables_body,
        out_shape=jax.ShapeDtypeStruct((_V, 1), jnp.float32),
    )(emb_m, W1, b1.reshape(1, -1), w2.reshape(-1, 1), b2.reshape(1, 1))


def _sc_body(ids_hbm, lt_hbm, pt_hbm, zeros_hbm,
             ml_hbm, pp_hbm, h16_hbm,
             ids_v, tab_v, ml_v, pp_v, h16_v,
             sem_misc, sem_ids, sem_out):
    wid = lax.axis_index("s") * _NC + lax.axis_index("c")

    cp_tab = pltpu.async_copy(lt_hbm, tab_v.at[pl.ds(0, _V)], sem_misc)
    cp_tab2 = pltpu.async_copy(pt_hbm, tab_v.at[pl.ds(_V, _V)], sem_misc)
    cp_zero = pltpu.async_copy(zeros_hbm, h16_v, sem_misc)
    row0 = wid * _ROWS_PER_W
    cp_ids = [
        pltpu.async_copy(ids_hbm.at[row0 + r], ids_v.at[pl.ds(r * _L, _L)],
                         sem_ids)
        for r in range(_ROWS_PER_W)
    ]
    cp_tab.wait()
    cp_tab2.wait()
    cp_zero.wait()
    for cp in cp_ids:
        cp.wait()

    lane_off = lax.iota(jnp.int32, _LANES) * _V
    ones = jnp.full((_LANES,), 1.0, jnp.float32)
    out_cps = []
    for r in range(_ROWS_PER_W):
        ib, hb = r * _L, r * _HB
        off = lane_off + hb

        def gather_body(i, _):
            for u in range(8):
                base = ib + (i * 8 + u) * _LANES
                idx = ids_v[pl.ds(base, _LANES)]
                ml_v[pl.ds(base, _LANES)] = plsc.load_gather(tab_v, [idx])
                pp_v[pl.ds(base, _LANES)] = plsc.load_gather(
                    tab_v, [idx + _V])
            return 0
        lax.fori_loop(0, _CHUNKS // 8, gather_body, 0)

        def hist_body(j, _):
            for u in range(8):
                base = ib + (j * 8 + u) * _LANES
                idx = ids_v[pl.ds(base, _LANES)]
                plsc.addupdate_scatter(h16_v, [idx + off], ones)
            return 0
        lax.fori_loop(0, _CHUNKS // 8, hist_body, 0)

        out_cps += [
            pltpu.async_copy(ml_v.at[pl.ds(ib, _L)], ml_hbm.at[row0 + r],
                             sem_out),
            pltpu.async_copy(pp_v.at[pl.ds(ib, _L)], pp_hbm.at[row0 + r],
                             sem_out),
            pltpu.async_copy(h16_v.at[pl.ds(hb, _HB)], h16_hbm.at[row0 + r],
                             sem_out),
        ]
    for cp in out_cps:
        cp.wait()


_sc_stage = functools.partial(
    pl.kernel,
    out_type=[
        jax.ShapeDtypeStruct((_B, _L), jnp.float32),
        jax.ShapeDtypeStruct((_B, _L), jnp.float32),
        jax.ShapeDtypeStruct((_B, _HB), jnp.float32),
    ],
    mesh=plsc.VectorSubcoreMesh(core_axis_name="c", subcore_axis_name="s"),
    compiler_params=pltpu.CompilerParams(needs_layout_passes=False),
    scratch_types=[
        pltpu.VMEM((_WL,), jnp.int32),
        pltpu.VMEM((2 * _V,), jnp.float32),
        pltpu.VMEM((_WL,), jnp.float32),
        pltpu.VMEM((_WL,), jnp.float32),
        pltpu.VMEM((_WH,), jnp.float32),
        pltpu.SemaphoreType.DMA,
        pltpu.SemaphoreType.DMA,
        pltpu.SemaphoreType.DMA,
    ],
)(_sc_body)


def _final_body(h16_ref, keep_ref, pc_ref, pr_ref, embv_ref, wcls_ref, bcls_ref,
                cls_ref, len_ref):
    hist = h16_ref[:, 0:_V]
    for l in range(1, _LANES):
        hist = hist + h16_ref[:, l * _V:(l + 1) * _V]
    kept = hist * keep_ref[...]
    count = jnp.sum(kept, axis=1, keepdims=True)
    pc = pc_ref[...]
    pr = pr_ref[...]
    iu = lax.broadcasted_iota(jnp.int32, (_V, _V), 0)
    iv = lax.broadcasted_iota(jnp.int32, (_V, _V), 1)
    before = jnp.where((pc > pr) | ((pc == pr) & (iu < iv)), 1.0, 0.0)
    cum = jnp.dot(hist, before, preferred_element_type=jnp.float32,
                  precision=lax.Precision.HIGHEST)
    topk_take = jnp.minimum(jnp.maximum(float(_MIN_KEPT) - cum, 0.0), hist)
    use_fb = count < float(_MIN_KEPT)
    w = jnp.where(use_fb, topk_take, kept)
    pooled_sum = jnp.dot(w, embv_ref[...], preferred_element_type=jnp.float32,
                         precision=lax.Precision.HIGHEST)
    lengths = jnp.where(use_fb, float(_MIN_KEPT), count)
    pooled = pooled_sum / jnp.maximum(lengths, 1.0)
    cls = jnp.dot(pooled, wcls_ref[...], preferred_element_type=jnp.float32)
    cls_ref[...] = cls + bcls_ref[0, 0]
    len_ref[...] = lengths.astype(jnp.int32)


def _finalize(h16, keep_row, prob_col, prob_row, emb_v_tail, W_cls_tail, b_cls):
    return pl.pallas_call(
        _final_body,
        out_shape=[
            jax.ShapeDtypeStruct((_B, 1), jnp.float32),
            jax.ShapeDtypeStruct((_B, 1), jnp.int32),
        ],
    )(h16, keep_row, prob_col, prob_row, emb_v_tail, W_cls_tail, b_cls)


def kernel(byte_ids, emb_m, W1, b1, w2, b2, emb_v, W_cls, b_cls):
    ids = jnp.asarray(byte_ids).astype(jnp.int32)
    logit_col = _masker_tables(emb_m, W1, b1, w2, b2)
    logit_tab = logit_col.reshape(_V)
    prob_tab = jax.nn.sigmoid(logit_tab)
    keep_row = (prob_tab > 0.5).astype(jnp.float32).reshape(1, _V)
    zeros_hb = jnp.zeros((_WH,), jnp.float32)

    mask_logits, prune_probs, h16 = _sc_stage(ids, logit_tab, prob_tab,
                                              zeros_hb)

    cls, lengths = _finalize(
        h16, keep_row, prob_tab.reshape(_V, 1),
        prob_tab.reshape(1, _V), emb_v, W_cls[2:, :], b_cls.reshape(1, 1))
    return mask_logits, prune_probs, cls, lengths.reshape(_B)


# speedup vs baseline: 1.2954x; 1.0832x over previous
import functools

import jax
import jax.numpy as jnp
from jax import lax
from jax.experimental import pallas as pl
from jax.experimental.pallas import tpu as pltpu
from jax.experimental.pallas import tpu_sc as plsc

_B, _L = 64, 2048
_V = 256
_MIN_KEPT = 4

_info = plsc.get_sparse_core_info()
_NC, _NS, _LANES = _info.num_cores, _info.num_subcores, _info.num_lanes
_NW = _NC * _NS
_ROWS_PER_W = _B // _NW
_CHUNKS = _L // _LANES
_HB = _LANES * _V
_WL = _ROWS_PER_W * _L
_WH = _ROWS_PER_W * _HB


def _tables_body(emb_ref, w1_ref, b1_ref, w2_ref, b2_ref, logit_ref):
    h = jnp.dot(emb_ref[...], w1_ref[...], preferred_element_type=jnp.float32)
    h = jnp.maximum(h + b1_ref[...], 0.0)
    logit = jnp.dot(h, w2_ref[...], preferred_element_type=jnp.float32)
    logit_ref[...] = logit + b2_ref[0, 0]


def _masker_tables(emb_m, W1, b1, w2, b2):
    return pl.pallas_call(
        _tables_body,
        out_shape=jax.ShapeDtypeStruct((_V, 1), jnp.float32),
    )(emb_m, W1, b1.reshape(1, -1), w2.reshape(-1, 1), b2.reshape(1, 1))


def _sc_body(ids_hbm, lt_hbm, pt_hbm, zeros_hbm,
             ml_hbm, pp_hbm, h16_hbm,
             ids_v, tab_v, ml_v, pp_v, h16_v,
             sem_misc, sem_ids, sem_out):
    wid = lax.axis_index("s") * _NC + lax.axis_index("c")

    cp_tab = pltpu.async_copy(lt_hbm, tab_v.at[pl.ds(0, _V)], sem_misc)
    cp_tab2 = pltpu.async_copy(pt_hbm, tab_v.at[pl.ds(_V, _V)], sem_misc)
    cp_zero = pltpu.async_copy(zeros_hbm, h16_v, sem_misc)
    row0 = wid * _ROWS_PER_W
    cp_ids = [
        pltpu.async_copy(ids_hbm.at[row0 + r], ids_v.at[pl.ds(r * _L, _L)],
                         sem_ids)
        for r in range(_ROWS_PER_W)
    ]
    cp_tab.wait()
    cp_tab2.wait()
    cp_zero.wait()
    for cp in cp_ids:
        cp.wait()

    lane_off = lax.iota(jnp.int32, _LANES) * _V
    ones = jnp.full((_LANES,), 1.0, jnp.float32)
    out_cps = []
    for r in range(_ROWS_PER_W):
        ib, hb = r * _L, r * _HB
        off = lane_off + hb

        def gather_body(i, _):
            bases = [ib + (i * 8 + u) * _LANES for u in range(8)]
            idxs = [ids_v[pl.ds(b, _LANES)] for b in bases]
            mls = [plsc.load_gather(tab_v, [idx]) for idx in idxs]
            pps = [plsc.load_gather(tab_v, [idx + _V]) for idx in idxs]
            for b, ml in zip(bases, mls):
                ml_v[pl.ds(b, _LANES)] = ml
            for b, pp in zip(bases, pps):
                pp_v[pl.ds(b, _LANES)] = pp
            return 0
        lax.fori_loop(0, _CHUNKS // 8, gather_body, 0)

        def hist_body(j, _):
            bases = [ib + (j * 8 + u) * _LANES for u in range(8)]
            idxs = [ids_v[pl.ds(b, _LANES)] + off for b in bases]
            for idx in idxs:
                plsc.addupdate_scatter(h16_v, [idx], ones)
            return 0
        lax.fori_loop(0, _CHUNKS // 8, hist_body, 0)

        out_cps += [
            pltpu.async_copy(ml_v.at[pl.ds(ib, _L)], ml_hbm.at[row0 + r],
                             sem_out),
            pltpu.async_copy(pp_v.at[pl.ds(ib, _L)], pp_hbm.at[row0 + r],
                             sem_out),
            pltpu.async_copy(h16_v.at[pl.ds(hb, _HB)], h16_hbm.at[row0 + r],
                             sem_out),
        ]
    for cp in out_cps:
        cp.wait()


_sc_stage = functools.partial(
    pl.kernel,
    out_type=[
        jax.ShapeDtypeStruct((_B, _L), jnp.float32),
        jax.ShapeDtypeStruct((_B, _L), jnp.float32),
        jax.ShapeDtypeStruct((_B, _HB), jnp.float32),
    ],
    mesh=plsc.VectorSubcoreMesh(core_axis_name="c", subcore_axis_name="s"),
    compiler_params=pltpu.CompilerParams(needs_layout_passes=False),
    scratch_types=[
        pltpu.VMEM((_WL,), jnp.int32),
        pltpu.VMEM((2 * _V,), jnp.float32),
        pltpu.VMEM((_WL,), jnp.float32),
        pltpu.VMEM((_WL,), jnp.float32),
        pltpu.VMEM((_WH,), jnp.float32),
        pltpu.SemaphoreType.DMA,
        pltpu.SemaphoreType.DMA,
        pltpu.SemaphoreType.DMA,
    ],
)(_sc_body)


def _final_body(h16_ref, keep_ref, pc_ref, pr_ref, embv_ref, wcls_ref, bcls_ref,
                cls_ref, len_ref):
    hist = h16_ref[:, 0:_V]
    for l in range(1, _LANES):
        hist = hist + h16_ref[:, l * _V:(l + 1) * _V]
    kept = hist * keep_ref[...]
    count = jnp.sum(kept, axis=1, keepdims=True)
    pc = pc_ref[...]
    pr = pr_ref[...]
    iu = lax.broadcasted_iota(jnp.int32, (_V, _V), 0)
    iv = lax.broadcasted_iota(jnp.int32, (_V, _V), 1)
    before = jnp.where((pc > pr) | ((pc == pr) & (iu < iv)), 1.0, 0.0)
    cum = jnp.dot(hist, before, preferred_element_type=jnp.float32,
                  precision=lax.Precision.HIGHEST)
    topk_take = jnp.minimum(jnp.maximum(float(_MIN_KEPT) - cum, 0.0), hist)
    use_fb = count < float(_MIN_KEPT)
    w = jnp.where(use_fb, topk_take, kept)
    pooled_sum = jnp.dot(w, embv_ref[...], preferred_element_type=jnp.float32,
                         precision=lax.Precision.HIGHEST)
    lengths = jnp.where(use_fb, float(_MIN_KEPT), count)
    pooled = pooled_sum / jnp.maximum(lengths, 1.0)
    cls = jnp.dot(pooled, wcls_ref[...], preferred_element_type=jnp.float32)
    cls_ref[...] = cls + bcls_ref[0, 0]
    len_ref[...] = lengths.astype(jnp.int32)


def _finalize(h16, keep_row, prob_col, prob_row, emb_v_tail, W_cls_tail, b_cls):
    return pl.pallas_call(
        _final_body,
        out_shape=[
            jax.ShapeDtypeStruct((_B, 1), jnp.float32),
            jax.ShapeDtypeStruct((_B, 1), jnp.int32),
        ],
    )(h16, keep_row, prob_col, prob_row, emb_v_tail, W_cls_tail, b_cls)


def kernel(byte_ids, emb_m, W1, b1, w2, b2, emb_v, W_cls, b_cls):
    ids = jnp.asarray(byte_ids).astype(jnp.int32)
    logit_col = _masker_tables(emb_m, W1, b1, w2, b2)
    logit_tab = logit_col.reshape(_V)
    prob_tab = jax.nn.sigmoid(logit_tab)
    keep_row = (prob_tab > 0.5).astype(jnp.float32).reshape(1, _V)
    zeros_hb = jnp.zeros((_WH,), jnp.float32)

    mask_logits, prune_probs, h16 = _sc_stage(ids, logit_tab, prob_tab,
                                              zeros_hb)

    cls, lengths = _finalize(
        h16, keep_row, prob_tab.reshape(_V, 1),
        prob_tab.reshape(1, _V), emb_v, W_cls[2:, :], b_cls.reshape(1, 1))
    return mask_logits, prune_probs, cls, lengths.reshape(_B)
